# baseline (speedup 1.0000x reference)
import functools
import jax
import jax.numpy as jnp
from jax import lax
from jax.experimental import pallas as pl
from jax.experimental.pallas import tpu as pltpu, tpu_sc as plsc

B = 256
N = 16384
NUM_WRITES = 4
NUM_READS = 8

CH = 2048
NW = 32
BPW = B // NW
CPB = N // CH
T = BPW * CPB
LANES = 16


def _sc_body(ww_hbm, fg_hbm, rw_hbm, pu_hbm, out_hbm,
             ww_v, rw_v, pu_v, out_v, fg_v,
             sem_in0, sem_in1, sem_out0, sem_out1):
    cid = lax.axis_index("c")
    sid = lax.axis_index("s")
    wid = sid * 2 + cid
    b0 = wid * BPW

    sem_in = (sem_in0, sem_in1)
    sem_out = (sem_out0, sem_out1)

    pltpu.sync_copy(fg_hbm, fg_v)

    def tile_bn(t):
        b = b0 + t // CPB
        n0 = (t % CPB) * CH
        return b, n0

    def start_in(t, j):
        b, n0 = tile_bn(t)
        pltpu.async_copy(ww_hbm.at[b, :, pl.ds(n0, CH)], ww_v.at[j], sem_in[j])
        pltpu.async_copy(rw_hbm.at[b, :, pl.ds(n0, CH)], rw_v.at[j], sem_in[j])
        pltpu.async_copy(pu_hbm.at[b, pl.ds(n0, CH)], pu_v.at[j], sem_in[j])

    def wait_in(t, j):
        b, n0 = tile_bn(t)
        pltpu.make_async_copy(ww_hbm.at[b, :, pl.ds(n0, CH)], ww_v.at[j], sem_in[j]).wait()
        pltpu.make_async_copy(rw_hbm.at[b, :, pl.ds(n0, CH)], rw_v.at[j], sem_in[j]).wait()
        pltpu.make_async_copy(pu_hbm.at[b, pl.ds(n0, CH)], pu_v.at[j], sem_in[j]).wait()

    def start_out(t, j):
        b, n0 = tile_bn(t)
        pltpu.async_copy(out_v.at[j], out_hbm.at[b, pl.ds(n0, CH)], sem_out[j])

    def wait_out(t, j):
        b, n0 = tile_bn(t)
        pltpu.make_async_copy(out_v.at[j], out_hbm.at[b, pl.ds(n0, CH)], sem_out[j]).wait()

    def compute(t, j):
        b, _ = tile_bn(t)
        fg_vec = fg_v[b, :]
        fgs = [fg_vec[r] for r in range(NUM_READS)]

        def step(i, carry):
            sl = pl.ds(i * LANES, LANES)
            pu16 = pu_v[j, sl]
            p = (1.0 - ww_v[j, 0, sl]) * (1.0 - ww_v[j, 1, sl])
            p = p * ((1.0 - ww_v[j, 2, sl]) * (1.0 - ww_v[j, 3, sl]))
            u = 1.0 - (1.0 - pu16) * p
            for r in range(NUM_READS):
                u = u * (1.0 - fgs[r] * rw_v[j, r, sl])
            out_v[j, sl] = u
            return carry

        lax.fori_loop(0, CH // LANES, step, 0, unroll=4)

    start_in(0, 0)

    def outer(g, carry):
        for j in (0, 1):
            t = 2 * g + j

            @pl.when(t + 1 < T)
            def _():
                start_in(t + 1, 1 - j)

            wait_in(t, j)

            @pl.when(t >= 2)
            def _():
                wait_out(t - 2, j)

            compute(t, j)
            start_out(t, j)
        return carry

    lax.fori_loop(0, T // 2, outer, 0)

    wait_out(T - 2, 0)
    wait_out(T - 1, 1)


def kernel(write_weights, free_gate, read_weights, prev_usage):
    mesh = plsc.VectorSubcoreMesh(core_axis_name="c", subcore_axis_name="s")
    f = functools.partial(
        pl.kernel,
        mesh=mesh,
        out_type=jax.ShapeDtypeStruct((B, N), jnp.float32),
        scratch_types=[
            pltpu.VMEM((2, NUM_WRITES, CH), jnp.float32),
            pltpu.VMEM((2, NUM_READS, CH), jnp.float32),
            pltpu.VMEM((2, CH), jnp.float32),
            pltpu.VMEM((2, CH), jnp.float32),
            pltpu.VMEM((B, LANES), jnp.float32),
            pltpu.SemaphoreType.DMA,
            pltpu.SemaphoreType.DMA,
            pltpu.SemaphoreType.DMA,
            pltpu.SemaphoreType.DMA,
        ],
    )(_sc_body)
    fg_pad = jnp.pad(free_gate, ((0, 0), (0, LANES - NUM_READS)))
    return f(write_weights, fg_pad, read_weights, prev_usage)


# speedup vs baseline: 1.6594x; 1.6594x over previous
import functools
import jax
import jax.numpy as jnp
from jax import lax
from jax.experimental import pallas as pl
from jax.experimental.pallas import tpu as pltpu, tpu_sc as plsc

B = 256
N = 16384
NUM_WRITES = 4
NUM_READS = 8

CH = 2048
NW = 32
BPW = B // NW
CPB = N // CH
T = BPW * CPB
LANES = 16


def _sc_body(ww_hbm, fg_hbm, rw_hbm, pu_hbm, out_hbm,
             ww_v, rw_v, pu_v, out_v, fg_v,
             sem_in0, sem_in1, sem_out0, sem_out1):
    cid = lax.axis_index("c")
    sid = lax.axis_index("s")
    wid = sid * 2 + cid
    b0 = wid * BPW

    sem_in = (sem_in0, sem_in1)
    sem_out = (sem_out0, sem_out1)

    pltpu.sync_copy(fg_hbm, fg_v)

    def tile_bn(t):
        b = b0 + t // CPB
        n0 = (t % CPB) * CH
        return b, n0

    def start_in(t, j):
        b, n0 = tile_bn(t)
        pltpu.async_copy(ww_hbm.at[b, :, pl.ds(n0, CH)], ww_v.at[j], sem_in[j])
        pltpu.async_copy(rw_hbm.at[b, :, pl.ds(n0, CH)], rw_v.at[j], sem_in[j])
        pltpu.async_copy(pu_hbm.at[b, pl.ds(n0, CH)], pu_v.at[j], sem_in[j])

    def wait_in(t, j):
        b, n0 = tile_bn(t)
        pltpu.make_async_copy(ww_hbm.at[b, :, pl.ds(n0, CH)], ww_v.at[j], sem_in[j]).wait()
        pltpu.make_async_copy(rw_hbm.at[b, :, pl.ds(n0, CH)], rw_v.at[j], sem_in[j]).wait()
        pltpu.make_async_copy(pu_hbm.at[b, pl.ds(n0, CH)], pu_v.at[j], sem_in[j]).wait()

    def start_out(t, j):
        b, n0 = tile_bn(t)
        pltpu.async_copy(out_v.at[j], out_hbm.at[b, pl.ds(n0, CH)], sem_out[j])

    def wait_out(t, j):
        b, n0 = tile_bn(t)
        pltpu.make_async_copy(out_v.at[j], out_hbm.at[b, pl.ds(n0, CH)], sem_out[j]).wait()

    def compute(t, j):
        b, _ = tile_bn(t)
        fg_vec = fg_v[b, :]
        fgb = [jnp.broadcast_to(fg_vec[r], (LANES,)) for r in range(NUM_READS)]

        @plsc.parallel_loop(0, CH, step=LANES, unroll=8)
        def _loop(i):
            sl = pl.ds(i, LANES)
            pu16 = pu_v[j, sl]
            p = ((1.0 - ww_v[j, 0, sl]) * (1.0 - ww_v[j, 1, sl])) * (
                (1.0 - ww_v[j, 2, sl]) * (1.0 - ww_v[j, 3, sl]))
            q = 1.0 - (1.0 - pu16) * p
            ts = [1.0 - fgb[r] * rw_v[j, r, sl] for r in range(NUM_READS)]
            s01 = ts[0] * ts[1]
            s23 = ts[2] * ts[3]
            s45 = ts[4] * ts[5]
            s67 = ts[6] * ts[7]
            u = (s01 * s23) * (s45 * s67)
            out_v[j, sl] = q * u

    start_in(0, 0)

    def outer(g, carry):
        for j in (0, 1):
            t = 2 * g + j

            @pl.when(t + 1 < T)
            def _():
                start_in(t + 1, 1 - j)

            wait_in(t, j)

            @pl.when(t >= 2)
            def _():
                wait_out(t - 2, j)

            compute(t, j)
            start_out(t, j)
        return carry

    lax.fori_loop(0, T // 2, outer, 0)

    wait_out(T - 2, 0)
    wait_out(T - 1, 1)


def kernel(write_weights, free_gate, read_weights, prev_usage):
    mesh = plsc.VectorSubcoreMesh(core_axis_name="c", subcore_axis_name="s")
    f = functools.partial(
        pl.kernel,
        mesh=mesh,
        out_type=jax.ShapeDtypeStruct((B, N), jnp.float32),
        scratch_types=[
            pltpu.VMEM((2, NUM_WRITES, CH), jnp.float32),
            pltpu.VMEM((2, NUM_READS, CH), jnp.float32),
            pltpu.VMEM((2, CH), jnp.float32),
            pltpu.VMEM((2, CH), jnp.float32),
            pltpu.VMEM((B, LANES), jnp.float32),
            pltpu.SemaphoreType.DMA,
            pltpu.SemaphoreType.DMA,
            pltpu.SemaphoreType.DMA,
            pltpu.SemaphoreType.DMA,
        ],
    )(_sc_body)
    fg_pad = jnp.pad(free_gate, ((0, 0), (0, LANES - NUM_READS)))
    return f(write_weights, fg_pad, read_weights, prev_usage)


# speedup vs baseline: 1.8578x; 1.1195x over previous
import functools
import jax
import jax.numpy as jnp
from jax import lax
from jax.experimental import pallas as pl
from jax.experimental.pallas import tpu as pltpu, tpu_sc as plsc

B = 256
N = 16384
NUM_WRITES = 4
NUM_READS = 8
LANES = 16

B_TC = 128

B_BLK = 32
N_BLK = 2048


def _tc_body(ww_ref, fg_ref, rw_ref, pu_ref, out_ref):
    pu = pu_ref[...]
    p = (1.0 - ww_ref[:, 0, :]) * (1.0 - ww_ref[:, 1, :])
    p = p * (1.0 - ww_ref[:, 2, :]) * (1.0 - ww_ref[:, 3, :])
    usage = 1.0 - (1.0 - pu) * p
    fg = fg_ref[...]
    phi = usage
    for r in range(NUM_READS):
        phi = phi * (1.0 - fg[:, r:r + 1] * rw_ref[:, r, :])
    out_ref[...] = phi


def _tc_part(ww, fg, rw, pu):
    grid = (B_TC // B_BLK, N // N_BLK)
    return pl.pallas_call(
        _tc_body,
        grid=grid,
        in_specs=[
            pl.BlockSpec((B_BLK, NUM_WRITES, N_BLK), lambda i, j: (i, 0, j)),
            pl.BlockSpec((B_BLK, NUM_READS), lambda i, j: (i, 0)),
            pl.BlockSpec((B_BLK, NUM_READS, N_BLK), lambda i, j: (i, 0, j)),
            pl.BlockSpec((B_BLK, N_BLK), lambda i, j: (i, j)),
        ],
        out_specs=pl.BlockSpec((B_BLK, N_BLK), lambda i, j: (i, j)),
        out_shape=jax.ShapeDtypeStruct((B_TC, N), jnp.float32),
    )(ww, fg, rw, pu)


B_SC = B - B_TC
NW = 32
BPW = B_SC // NW
CH = 2048
CPB = N // CH
T = BPW * CPB


def _sc_body(ww_hbm, fg_hbm, rw_hbm, pu_hbm, out_hbm,
             ww_v, rw_v, pu_v, out_v, fg_v,
             sem_in0, sem_in1, sem_out0, sem_out1):
    cid = lax.axis_index("c")
    sid = lax.axis_index("s")
    wid = sid * 2 + cid
    b0 = wid * BPW

    sem_in = (sem_in0, sem_in1)
    sem_out = (sem_out0, sem_out1)

    pltpu.sync_copy(fg_hbm, fg_v)

    def tile_bn(t):
        b = b0 + t // CPB
        n0 = (t % CPB) * CH
        return b, n0

    def start_in(t, j):
        b, n0 = tile_bn(t)
        bg = b + B_TC
        pltpu.async_copy(ww_hbm.at[bg, :, pl.ds(n0, CH)], ww_v.at[j], sem_in[j])
        pltpu.async_copy(rw_hbm.at[bg, :, pl.ds(n0, CH)], rw_v.at[j], sem_in[j])
        pltpu.async_copy(pu_hbm.at[bg, pl.ds(n0, CH)], pu_v.at[j], sem_in[j])

    def wait_in(t, j):
        b, n0 = tile_bn(t)
        bg = b + B_TC
        pltpu.make_async_copy(ww_hbm.at[bg, :, pl.ds(n0, CH)], ww_v.at[j], sem_in[j]).wait()
        pltpu.make_async_copy(rw_hbm.at[bg, :, pl.ds(n0, CH)], rw_v.at[j], sem_in[j]).wait()
        pltpu.make_async_copy(pu_hbm.at[bg, pl.ds(n0, CH)], pu_v.at[j], sem_in[j]).wait()

    def start_out(t, j):
        b, n0 = tile_bn(t)
        pltpu.async_copy(out_v.at[j], out_hbm.at[b + B_TC, pl.ds(n0, CH)], sem_out[j])

    def wait_out(t, j):
        b, n0 = tile_bn(t)
        pltpu.make_async_copy(out_v.at[j], out_hbm.at[b + B_TC, pl.ds(n0, CH)], sem_out[j]).wait()

    def compute(t, j):
        b, _ = tile_bn(t)
        fg_vec = fg_v[b + B_TC, :]
        fgb = [jnp.broadcast_to(fg_vec[r], (LANES,)) for r in range(NUM_READS)]

        @plsc.parallel_loop(0, CH, step=LANES, unroll=8)
        def _loop(i):
            sl = pl.ds(i, LANES)
            pu16 = pu_v[j, sl]
            p = ((1.0 - ww_v[j, 0, sl]) * (1.0 - ww_v[j, 1, sl])) * (
                (1.0 - ww_v[j, 2, sl]) * (1.0 - ww_v[j, 3, sl]))
            q = 1.0 - (1.0 - pu16) * p
            ts = [1.0 - fgb[r] * rw_v[j, r, sl] for r in range(NUM_READS)]
            u = ((ts[0] * ts[1]) * (ts[2] * ts[3])) * (
                (ts[4] * ts[5]) * (ts[6] * ts[7]))
            out_v[j, sl] = q * u

    start_in(0, 0)

    def outer(g, carry):
        for j in (0, 1):
            t = 2 * g + j

            @pl.when(t + 1 < T)
            def _():
                start_in(t + 1, 1 - j)

            wait_in(t, j)

            @pl.when(t >= 2)
            def _():
                wait_out(t - 2, j)

            compute(t, j)
            start_out(t, j)
        return carry

    lax.fori_loop(0, T // 2, outer, 0)

    wait_out(T - 2, 0)
    wait_out(T - 1, 1)


def _sc_part(ww, fg_pad, rw, pu):
    mesh = plsc.VectorSubcoreMesh(core_axis_name="c", subcore_axis_name="s")
    f = functools.partial(
        pl.kernel,
        mesh=mesh,
        out_type=jax.ShapeDtypeStruct((B, N), jnp.float32),
        scratch_types=[
            pltpu.VMEM((2, NUM_WRITES, CH), jnp.float32),
            pltpu.VMEM((2, NUM_READS, CH), jnp.float32),
            pltpu.VMEM((2, CH), jnp.float32),
            pltpu.VMEM((2, CH), jnp.float32),
            pltpu.VMEM((B, LANES), jnp.float32),
            pltpu.SemaphoreType.DMA,
            pltpu.SemaphoreType.DMA,
            pltpu.SemaphoreType.DMA,
            pltpu.SemaphoreType.DMA,
        ],
    )(_sc_body)
    return f(ww, fg_pad, rw, pu)


def kernel(write_weights, free_gate, read_weights, prev_usage):
    fg_pad = jnp.pad(free_gate, ((0, 0), (0, LANES - NUM_READS)))
    out_sc = _sc_part(write_weights, fg_pad, read_weights, prev_usage)
    out_tc = _tc_part(write_weights, free_gate, read_weights, prev_usage)
    return lax.dynamic_update_slice(out_sc, out_tc, (0, 0))
